# Initial kernel scaffold; baseline (speedup 1.0000x reference)
import jax
import jax.numpy as jnp
from jax.experimental import pallas as pl


def kernel(x, batch, dataset_name, W_shared, b_shared, W_head, b_head):
    raise NotImplementedError("write your pallas kernel here")


# speedup vs baseline: 4.8004x; 4.8004x over previous
import functools

import jax
import jax.numpy as jnp
from jax import lax
from jax.experimental import pallas as pl
from jax.experimental.pallas import tpu as pltpu
from jax.experimental.pallas import tpu_sc as plsc

N = 100000
D = 128
G = 1024
B = 8
H = 128
HEAD_DIM = 1

NC = 2
NS = 16
NW = NC * NS
CH = 128
NFULL = N // CH
TAIL = N - NFULL * CH
ROWS_PER_TILE = G // NS


def _pool_body(x_hbm, batch_hbm, sums_hbm, cnt_hbm,
               buf, idx, ones, zbuf, tbuf, tidx, acc_sh, cnt_sh):
    c = lax.axis_index("c")
    s = lax.axis_index("s")
    wid = s * NC + c

    def _fill(r, _):
        for k in range(D // 16):
            ones[r, pl.ds(k * 16, 16)] = jnp.full((16,), 1.0, jnp.float32)

        @pl.when(r < ROWS_PER_TILE)
        def _z():
            for k in range(D // 16):
                zbuf[r, pl.ds(k * 16, 16)] = jnp.zeros((16,), jnp.float32)

        return 0

    lax.fori_loop(0, CH, _fill, 0)

    pltpu.sync_copy(zbuf.at[pl.ds(0, ROWS_PER_TILE)],
                    acc_sh.at[pl.ds(s * ROWS_PER_TILE, ROWS_PER_TILE)])
    pltpu.sync_copy(zbuf.at[pl.ds(0, ROWS_PER_TILE)],
                    cnt_sh.at[pl.ds(s * ROWS_PER_TILE, ROWS_PER_TILE)])
    plsc.subcore_barrier()

    nj = (NFULL - wid + NW - 1) // NW

    def _chunk(j, _):
        base = (wid + j * NW) * CH
        pltpu.sync_copy(batch_hbm.at[pl.ds(base, CH)], idx)
        pltpu.sync_copy(x_hbm.at[pl.ds(base, CH)], buf)
        pltpu.sync_copy(buf, acc_sh.at[idx], add=True)
        pltpu.sync_copy(ones, cnt_sh.at[idx], add=True)
        return 0

    lax.fori_loop(0, nj, _chunk, 0)

    @pl.when(wid == NW - 1)
    def _tail():
        pltpu.sync_copy(batch_hbm.at[pl.ds(NFULL * CH, TAIL)], tidx)
        pltpu.sync_copy(x_hbm.at[pl.ds(NFULL * CH, TAIL)], tbuf)
        pltpu.sync_copy(tbuf, acc_sh.at[tidx], add=True)
        pltpu.sync_copy(ones.at[pl.ds(0, TAIL)], cnt_sh.at[tidx], add=True)

    plsc.subcore_barrier()

    pltpu.sync_copy(acc_sh.at[pl.ds(s * ROWS_PER_TILE, ROWS_PER_TILE)],
                    sums_hbm.at[c, pl.ds(s * ROWS_PER_TILE, ROWS_PER_TILE)])
    pltpu.sync_copy(cnt_sh.at[pl.ds(s * ROWS_PER_TILE, ROWS_PER_TILE)],
                    cnt_hbm.at[c, pl.ds(s * ROWS_PER_TILE, ROWS_PER_TILE)])


@functools.lru_cache(maxsize=1)
def _get_pool():
  return pl.kernel(
    _pool_body,
    out_type=(
        jax.ShapeDtypeStruct((NC, G, D), jnp.float32),
        jax.ShapeDtypeStruct((NC, G, D), jnp.float32),
    ),
    mesh=plsc.VectorSubcoreMesh(core_axis_name="c", subcore_axis_name="s",
                                num_cores=NC, num_subcores=NS),
    scratch_types=[
        pltpu.VMEM((CH, D), jnp.float32),
        pltpu.VMEM((CH,), jnp.int32),
        pltpu.VMEM((CH, D), jnp.float32),
        pltpu.VMEM((ROWS_PER_TILE, D), jnp.float32),
        pltpu.VMEM((TAIL, D), jnp.float32),
        pltpu.VMEM((TAIL,), jnp.int32),
        pltpu.VMEM_SHARED((G, D), jnp.float32),
        pltpu.VMEM_SHARED((G, D), jnp.float32),
    ],
  )


def _dense_body(sums_ref, cnt_ref, ds_ref, Ws_ref, bs_ref, Wh_ref, bh_ref,
                head_ref, var_ref):
    sums = sums_ref[0] + sums_ref[1]
    counts = cnt_ref[0, :, 0:1] + cnt_ref[1, :, 0:1]
    xg = sums / jnp.maximum(counts, 1.0)
    ds = ds_ref[...]

    out = jnp.zeros((G, 2 * HEAD_DIM), jnp.float32)
    for b in range(B):
        h = jnp.dot(xg, Ws_ref[b], preferred_element_type=jnp.float32)
        h = jnp.maximum(h + bs_ref[b][None, :], 0.0)
        o = jnp.dot(h, Wh_ref[b], preferred_element_type=jnp.float32)
        o = o + bh_ref[b][None, :]
        out = jnp.where(ds == b, o, out)

    head_ref[...] = out[:, :HEAD_DIM]
    var_ref[...] = out[:, HEAD_DIM:] ** 2


_dense = pl.pallas_call(
    _dense_body,
    out_shape=(
        jax.ShapeDtypeStruct((G, HEAD_DIM), jnp.float32),
        jax.ShapeDtypeStruct((G, HEAD_DIM), jnp.float32),
    ),
)


@jax.jit
def kernel(x, batch, dataset_name, W_shared, b_shared, W_head, b_head):
    sums, cnt = _get_pool()(x, batch)
    head, var = _dense(sums, cnt, dataset_name, W_shared, b_shared,
                       W_head, b_head)
    return (head, var)


# speedup vs baseline: 6.6122x; 1.3774x over previous
import functools

import jax
import jax.numpy as jnp
from jax import lax
from jax.experimental import pallas as pl
from jax.experimental.pallas import tpu as pltpu
from jax.experimental.pallas import tpu_sc as plsc

N = 100000
D = 128
G = 1024
B = 8
H = 128
HEAD_DIM = 1

NC = 2
NS = 16
NW = NC * NS
CH = 256
NFULL = N // CH
TAILA = 128
TAILB = 32
ROWS_PER_TILE = G // NS


def _pool_body(x_hbm, batch_hbm, sums_hbm, cnt_hbm,
               buf0, buf1, idx0, idx1, ones, zbuf, tbufa, tbufb, tidxa, tidxb,
               acc_sh, cnt_sh, sem0, sem1, isem0, isem1):
    c = lax.axis_index("c")
    s = lax.axis_index("s")
    wid = s * NC + c

    def _fill(r, _):
        for k in range(D // 16):
            ones[r, pl.ds(k * 16, 16)] = jnp.full((16,), 1.0, jnp.float32)

        @pl.when(r < ROWS_PER_TILE)
        def _z():
            for k in range(D // 16):
                zbuf[r, pl.ds(k * 16, 16)] = jnp.zeros((16,), jnp.float32)

        return 0

    lax.fori_loop(0, 128, _fill, 0)

    pltpu.sync_copy(zbuf, acc_sh.at[pl.ds(s * ROWS_PER_TILE, ROWS_PER_TILE)])
    pltpu.sync_copy(zbuf, cnt_sh.at[pl.ds(s * ROWS_PER_TILE, ROWS_PER_TILE)])
    plsc.subcore_barrier()

    nj = (NFULL - wid + NW - 1) // NW
    bufs = ((buf0, idx0, sem0, isem0), (buf1, idx1, sem1, isem1))

    def _issue(j, bufp, idxp, semp, isemp):
        base = (wid + j * NW) * CH
        pltpu.async_copy(x_hbm.at[pl.ds(base, CH)], bufp, semp)
        pltpu.async_copy(batch_hbm.at[pl.ds(base, 128)], idxp.at[0], isemp)
        pltpu.async_copy(batch_hbm.at[pl.ds(base + 128, 128)], idxp.at[1],
                         isemp)

    @pl.when(nj > 0)
    def _prologue():
        _issue(0, buf0, idx0, sem0, isem0)

    def _outer(j2, _):
        for b2 in (0, 1):
            j = j2 * 2 + b2
            bufp, idxp, semp, isemp = bufs[b2]
            bufn, idxn, semn, isemn = bufs[1 - b2]

            @pl.when(j < nj)
            def _do():
                @pl.when(j + 1 < nj)
                def _next():
                    _issue(j + 1, bufn, idxn, semn, isemn)

                base = (wid + j * NW) * CH
                pltpu.make_async_copy(x_hbm.at[pl.ds(base, CH)], bufp,
                                      semp).wait()
                pltpu.make_async_copy(batch_hbm.at[pl.ds(base, 128)],
                                      idxp.at[0], isemp).wait()
                pltpu.make_async_copy(batch_hbm.at[pl.ds(base, 128)],
                                      idxp.at[1], isemp).wait()
                for h in (0, 1):
                    pltpu.sync_copy(bufp.at[pl.ds(h * 128, 128)],
                                    acc_sh.at[idxp.at[h]], add=True)
                    pltpu.sync_copy(ones, cnt_sh.at[idxp.at[h]], add=True)

        return 0

    lax.fori_loop(0, (nj + 1) // 2, _outer, 0)

    @pl.when(wid == NW - 1)
    def _tail():
        base = NFULL * CH
        pltpu.sync_copy(batch_hbm.at[pl.ds(base, TAILA)], tidxa.at[0])
        pltpu.sync_copy(x_hbm.at[pl.ds(base, TAILA)], tbufa)
        pltpu.sync_copy(tbufa, acc_sh.at[tidxa.at[0]], add=True)
        pltpu.sync_copy(ones, cnt_sh.at[tidxa.at[0]], add=True)
        pltpu.sync_copy(batch_hbm.at[pl.ds(base + TAILA, TAILB)], tidxb.at[0])
        pltpu.sync_copy(x_hbm.at[pl.ds(base + TAILA, TAILB)], tbufb)
        pltpu.sync_copy(tbufb, acc_sh.at[tidxb.at[0]], add=True)
        pltpu.sync_copy(ones.at[pl.ds(0, TAILB)], cnt_sh.at[tidxb.at[0]],
                        add=True)

    plsc.subcore_barrier()

    pltpu.sync_copy(acc_sh.at[pl.ds(s * ROWS_PER_TILE, ROWS_PER_TILE)],
                    sums_hbm.at[c, pl.ds(s * ROWS_PER_TILE, ROWS_PER_TILE)])
    pltpu.sync_copy(cnt_sh.at[pl.ds(s * ROWS_PER_TILE, ROWS_PER_TILE)],
                    cnt_hbm.at[c, pl.ds(s * ROWS_PER_TILE, ROWS_PER_TILE)])


@functools.lru_cache(maxsize=1)
def _get_pool():
  return pl.kernel(
    _pool_body,
    out_type=(
        jax.ShapeDtypeStruct((NC, G, D), jnp.float32),
        jax.ShapeDtypeStruct((NC, G, D), jnp.float32),
    ),
    mesh=plsc.VectorSubcoreMesh(core_axis_name="c", subcore_axis_name="s",
                                num_cores=NC, num_subcores=NS),
    scratch_types=[
        pltpu.VMEM((CH, D), jnp.float32),
        pltpu.VMEM((CH, D), jnp.float32),
        pltpu.VMEM((2, 128), jnp.int32),
        pltpu.VMEM((2, 128), jnp.int32),
        pltpu.VMEM((128, D), jnp.float32),
        pltpu.VMEM((ROWS_PER_TILE, D), jnp.float32),
        pltpu.VMEM((TAILA, D), jnp.float32),
        pltpu.VMEM((TAILB, D), jnp.float32),
        pltpu.VMEM((1, TAILA), jnp.int32),
        pltpu.VMEM((1, TAILB), jnp.int32),
        pltpu.VMEM_SHARED((G, D), jnp.float32),
        pltpu.VMEM_SHARED((G, D), jnp.float32),
        pltpu.SemaphoreType.DMA,
        pltpu.SemaphoreType.DMA,
        pltpu.SemaphoreType.DMA,
        pltpu.SemaphoreType.DMA,
    ],
  )


def _dense_body(sums_ref, cnt_ref, ds_ref, Ws_ref, bs_ref, Wh_ref, bh_ref,
                head_ref, var_ref):
    sums = sums_ref[0] + sums_ref[1]
    counts = cnt_ref[0, :, 0:1] + cnt_ref[1, :, 0:1]
    xg = sums / jnp.maximum(counts, 1.0)
    ds = ds_ref[...]

    out = jnp.zeros((G, 2 * HEAD_DIM), jnp.float32)
    for b in range(B):
        h = jnp.dot(xg, Ws_ref[b], preferred_element_type=jnp.float32)
        h = jnp.maximum(h + bs_ref[b][None, :], 0.0)
        o = jnp.dot(h, Wh_ref[b], preferred_element_type=jnp.float32)
        o = o + bh_ref[b][None, :]
        out = jnp.where(ds == b, o, out)

    head_ref[...] = out[:, :HEAD_DIM]
    var_ref[...] = out[:, HEAD_DIM:] ** 2


_dense = pl.pallas_call(
    _dense_body,
    out_shape=(
        jax.ShapeDtypeStruct((G, HEAD_DIM), jnp.float32),
        jax.ShapeDtypeStruct((G, HEAD_DIM), jnp.float32),
    ),
)


@jax.jit
def kernel(x, batch, dataset_name, W_shared, b_shared, W_head, b_head):
    sums, cnt = _get_pool()(x, batch)
    head, var = _dense(sums, cnt, dataset_name, W_shared, b_shared,
                       W_head, b_head)
    return (head, var)
